# baseline (speedup 1.0000x reference)
import functools

import jax
import jax.numpy as jnp
from jax import lax
from jax.experimental import pallas as pl
from jax.experimental.pallas import tpu as pltpu
from jax.experimental.pallas import tpu_sc as plsc

N = 10000
D = 128
E = 320000
NCORES = 2
NSUB = 16
NTILES = NCORES * NSUB
CHUNK = 64
CHUNKS_PER_TILE = 256
NSEG = 4
SEGCH = CHUNKS_PER_TILE // NSEG
NBUF = 3
EDGES_PER_TILE = CHUNK * CHUNKS_PER_TILE
E_PAD = EDGES_PER_TILE * NTILES
E_CORE0 = NSUB * EDGES_PER_TILE
E_CORE1 = E - E_CORE0
ACC_ROWS = 10112
ROWS_PER_TILE = ACC_ROWS // NSUB
_WINDOWS = tuple(min(k * CHUNK, ROWS_PER_TILE - CHUNK) for k in range(10))
DEG_W = 128

_MESH = plsc.VectorSubcoreMesh(core_axis_name="c", subcore_axis_name="s")


@functools.partial(
    pl.kernel,
    mesh=_MESH,
    out_type=jax.ShapeDtypeStruct((NCORES * ACC_ROWS, DEG_W), jnp.float32),
    scratch_types=[
        pltpu.VMEM((CHUNKS_PER_TILE, CHUNK), jnp.int32),
        pltpu.VMEM((CHUNK, DEG_W), jnp.float32),
        pltpu.VMEM((CHUNK, DEG_W), jnp.float32),
        pltpu.VMEM_SHARED((ACC_ROWS, DEG_W), jnp.float32),
        pltpu.SemaphoreType.DMA,
    ],
)
def _degree_sc(dst_hbm, ones_hbm, zeros_hbm, out_hbm, didx, ones_v, wb_v, acc, sem):
    cid = lax.axis_index("c")
    sid = lax.axis_index("s")
    tid = cid * NSUB + sid
    pltpu.sync_copy(zeros_hbm, wb_v)
    for w in _WINDOWS:
        pltpu.sync_copy(wb_v, acc.at[pl.ds(sid * ROWS_PER_TILE + w, CHUNK)])
    pltpu.sync_copy(ones_hbm, ones_v)
    pltpu.sync_copy(dst_hbm.at[pl.ds(tid * CHUNKS_PER_TILE, CHUNKS_PER_TILE)], didx)
    plsc.subcore_barrier()

    GROUP = 8
    for g in range(CHUNKS_PER_TILE // GROUP):
        descs = [
            pltpu.async_copy(ones_v, acc.at[didx.at[g * GROUP + j]], sem, add=True)
            for j in range(GROUP)
        ]
        for desc in descs:
            desc.wait()
    plsc.subcore_barrier()
    for w in _WINDOWS:
        r = sid * ROWS_PER_TILE + w
        pltpu.sync_copy(acc.at[pl.ds(r, CHUNK)], wb_v)
        pltpu.sync_copy(wb_v, out_hbm.at[pl.ds(cid * ACC_ROWS + r, CHUNK)])


@functools.partial(
    pl.kernel,
    mesh=_MESH,
    out_type=jax.ShapeDtypeStruct((NCORES * ACC_ROWS, D), jnp.float32),
    scratch_types=[
        pltpu.VMEM((SEGCH, CHUNK), jnp.int32),
        pltpu.VMEM((SEGCH, CHUNK), jnp.int32),
        pltpu.VMEM((NBUF, CHUNK, D), jnp.float32),
        pltpu.VMEM_SHARED((ACC_ROWS, D), jnp.float32),
        pltpu.SemaphoreType.DMA,
        pltpu.SemaphoreType.DMA,
        pltpu.SemaphoreType.DMA,
    ],
)
def _scatter_sc(u_hbm, src_hbm, dst_hbm, zeros_hbm, out_hbm,
                sidx, didx, ring, acc, sem0, sem1, sem2):
    cid = lax.axis_index("c")
    sid = lax.axis_index("s")
    tid = cid * NSUB + sid
    pltpu.sync_copy(zeros_hbm, ring.at[0])
    for w in _WINDOWS:
        pltpu.sync_copy(ring.at[0], acc.at[pl.ds(sid * ROWS_PER_TILE + w, CHUNK)])
    plsc.subcore_barrier()

    sems = (sem0, sem1, sem2)
    for h in range(NSEG):
        hb = tid * CHUNKS_PER_TILE + h * SEGCH
        pltpu.sync_copy(src_hbm.at[pl.ds(hb, SEGCH)], sidx)
        pltpu.sync_copy(dst_hbm.at[pl.ds(hb, SEGCH)], didx)
        gathers = [
            pltpu.async_copy(u_hbm.at[sidx.at[j]], ring.at[j], sems[j])
            for j in range(NBUF)
        ]
        for i in range(SEGCH):
            p = i % NBUF
            gathers[p].wait()
            pltpu.sync_copy(ring.at[p], acc.at[didx.at[i]], add=True)
            if i + NBUF < SEGCH:
                gathers[p] = pltpu.async_copy(
                    u_hbm.at[sidx.at[i + NBUF]], ring.at[p], sems[p])
    plsc.subcore_barrier()
    for w in _WINDOWS:
        r = sid * ROWS_PER_TILE + w
        pltpu.sync_copy(acc.at[pl.ds(r, CHUNK)], ring.at[0])
        pltpu.sync_copy(ring.at[0], out_hbm.at[pl.ds(cid * ACC_ROWS + r, CHUNK)])


BLK = 1000


def _stage_a_body(x_ref, wfc_ref, bfc_ref, w1_ref, deg_ref, u1_ref, dinv_ref):
    d = deg_ref[...]
    deg = d[0] + d[1] + 1.0

    dinvb = jnp.broadcast_to(lax.rsqrt(deg[:, 0:1]), (BLK, D))
    h0 = jnp.maximum(
        jnp.dot(x_ref[...], wfc_ref[...], preferred_element_type=jnp.float32)
        + bfc_ref[...], 0.0)
    u1_ref[...] = jnp.dot(h0, w1_ref[...],
                          preferred_element_type=jnp.float32) * dinvb
    dinv_ref[...] = dinvb


_stage_a = pl.pallas_call(
    _stage_a_body,
    grid=(N // BLK,),
    in_specs=[
        pl.BlockSpec((BLK, D), lambda i: (i, 0)),
        pl.BlockSpec((D, D), lambda i: (0, 0)),
        pl.BlockSpec((1, D), lambda i: (0, 0)),
        pl.BlockSpec((D, D), lambda i: (0, 0)),
        pl.BlockSpec((NCORES, BLK, DEG_W), lambda i: (0, i, 0)),
    ],
    out_specs=[pl.BlockSpec((BLK, D), lambda i: (i, 0))] * 2,
    out_shape=[jax.ShapeDtypeStruct((N, D), jnp.float32)] * 2,
)


def _stage_b_body(s_ref, u1_ref, dinv_ref, b1_ref, w2_ref, u2_ref):
    s = s_ref[...]
    dinvb = dinv_ref[...]
    h1 = jnp.maximum((s[0] + s[1] + u1_ref[...]) * dinvb + b1_ref[...], 0.0)
    u2_ref[...] = jnp.dot(h1, w2_ref[...],
                          preferred_element_type=jnp.float32) * dinvb


_stage_b = pl.pallas_call(
    _stage_b_body,
    grid=(N // BLK,),
    in_specs=[
        pl.BlockSpec((NCORES, BLK, D), lambda i: (0, i, 0)),
        pl.BlockSpec((BLK, D), lambda i: (i, 0)),
        pl.BlockSpec((BLK, D), lambda i: (i, 0)),
        pl.BlockSpec((1, D), lambda i: (0, 0)),
        pl.BlockSpec((D, D), lambda i: (0, 0)),
    ],
    out_specs=pl.BlockSpec((BLK, D), lambda i: (i, 0)),
    out_shape=jax.ShapeDtypeStruct((N, D), jnp.float32),
)


def _stage_c_body(s_ref, u2_ref, dinv_ref, b2_ref, out_ref):
    s = s_ref[...]
    out_ref[...] = (s[0] + s[1] + u2_ref[...]) * dinv_ref[...] + b2_ref[...]


_stage_c = pl.pallas_call(
    _stage_c_body,
    grid=(N // BLK,),
    in_specs=[
        pl.BlockSpec((NCORES, BLK, D), lambda i: (0, i, 0)),
        pl.BlockSpec((BLK, D), lambda i: (i, 0)),
        pl.BlockSpec((BLK, D), lambda i: (i, 0)),
        pl.BlockSpec((1, D), lambda i: (0, 0)),
    ],
    out_specs=pl.BlockSpec((BLK, D), lambda i: (i, 0)),
    out_shape=jax.ShapeDtypeStruct((N, D), jnp.float32),
)


def kernel(x, edge_index, W_fc, b_fc, W1, b1, W2, b2):
    src = edge_index[0].astype(jnp.int32)
    dst = edge_index[1].astype(jnp.int32)
    pad = E_PAD - E
    dum = N + (jnp.arange(pad, dtype=jnp.int32) % NSUB)
    src_p = jnp.concatenate([src, jnp.zeros((pad,), jnp.int32)])
    src_p = src_p.reshape(NTILES * CHUNKS_PER_TILE, CHUNK)
    dst_p = jnp.concatenate([dst, dum])
    dst_p = dst_p.reshape(NTILES * CHUNKS_PER_TILE, CHUNK)
    ones128 = jnp.ones((CHUNK, DEG_W), jnp.float32)
    zeros128 = jnp.zeros((CHUNK, D), jnp.float32)

    deg = _degree_sc(dst_p, ones128, zeros128).reshape(NCORES, ACC_ROWS, DEG_W)
    u1, dinvb = _stage_a(x, W_fc, b_fc.reshape(1, D), W1, deg)
    s1 = _scatter_sc(u1, src_p, dst_p, zeros128).reshape(NCORES, ACC_ROWS, D)
    u2 = _stage_b(s1, u1, dinvb, b1.reshape(1, D), W2)
    s2 = _scatter_sc(u2, src_p, dst_p, zeros128).reshape(NCORES, ACC_ROWS, D)
    out = _stage_c(s2, u2, dinvb, b2.reshape(1, D))
    return out


# speedup vs baseline: 14.8267x; 14.8267x over previous
import functools

import jax
import jax.numpy as jnp
from jax import lax
from jax.experimental import pallas as pl
from jax.experimental.pallas import tpu as pltpu
from jax.experimental.pallas import tpu_sc as plsc

N = 10000
D = 128
E = 320000
NCORES = 2
NSUB = 16
NTILES = NCORES * NSUB
CHUNK = 64
CHUNKS_PER_TILE = 256
NSEG = 4
SEGCH = CHUNKS_PER_TILE // NSEG
NBUF = 3
EDGES_PER_TILE = CHUNK * CHUNKS_PER_TILE
E_PAD = EDGES_PER_TILE * NTILES
FAST_CORE = 0
E_BIG = NSUB * EDGES_PER_TILE
E_SMALL = E - E_BIG
SMALL_SLOTS = NSUB * SEGCH * CHUNK
ACC_ROWS = 10112
ROWS_PER_TILE = ACC_ROWS // NSUB
_WINDOWS = tuple(min(k * CHUNK, ROWS_PER_TILE - CHUNK) for k in range(10))
DEG_W = 128

_MESH = plsc.VectorSubcoreMesh(core_axis_name="c", subcore_axis_name="s")


@functools.partial(
    pl.kernel,
    mesh=_MESH,
    out_type=jax.ShapeDtypeStruct((NCORES * ACC_ROWS, DEG_W), jnp.float32),
    scratch_types=[
        pltpu.VMEM((CHUNKS_PER_TILE, CHUNK), jnp.int32),
        pltpu.VMEM((CHUNK, DEG_W), jnp.float32),
        pltpu.VMEM((CHUNK, DEG_W), jnp.float32),
        pltpu.VMEM_SHARED((ACC_ROWS, DEG_W), jnp.float32),
        pltpu.SemaphoreType.DMA,
    ],
)
def _degree_sc(dst_hbm, ones_hbm, zeros_hbm, out_hbm, didx, ones_v, wb_v, acc, sem):
    cid = lax.axis_index("c")
    sid = lax.axis_index("s")
    tid = cid * NSUB + sid
    pltpu.sync_copy(zeros_hbm, wb_v)
    for w in _WINDOWS:
        pltpu.sync_copy(wb_v, acc.at[pl.ds(sid * ROWS_PER_TILE + w, CHUNK)])
    pltpu.sync_copy(ones_hbm, ones_v)
    pltpu.sync_copy(dst_hbm.at[pl.ds(tid * CHUNKS_PER_TILE, CHUNKS_PER_TILE)], didx)
    plsc.subcore_barrier()

    GROUP = 8

    def run_groups(lo, hi):
        for g in range(lo, hi):
            descs = [
                pltpu.async_copy(ones_v, acc.at[didx.at[g * GROUP + j]], sem,
                                 add=True)
                for j in range(GROUP)
            ]
            for desc in descs:
                desc.wait()

    run_groups(0, SEGCH // GROUP)
    @pl.when(cid == FAST_CORE)
    def _():
        run_groups(SEGCH // GROUP, CHUNKS_PER_TILE // GROUP)
    plsc.subcore_barrier()
    for w in _WINDOWS:
        r = sid * ROWS_PER_TILE + w
        pltpu.sync_copy(acc.at[pl.ds(r, CHUNK)], wb_v)
        pltpu.sync_copy(wb_v, out_hbm.at[pl.ds(cid * ACC_ROWS + r, CHUNK)])


@functools.partial(
    pl.kernel,
    mesh=_MESH,
    out_type=jax.ShapeDtypeStruct((NCORES * ACC_ROWS, D), jnp.float32),
    scratch_types=[
        pltpu.VMEM((SEGCH, CHUNK), jnp.int32),
        pltpu.VMEM((SEGCH, CHUNK), jnp.int32),
        pltpu.VMEM((NBUF, CHUNK, D), jnp.float32),
        pltpu.VMEM_SHARED((ACC_ROWS, D), jnp.float32),
        pltpu.SemaphoreType.DMA,
        pltpu.SemaphoreType.DMA,
        pltpu.SemaphoreType.DMA,
    ],
)
def _scatter_sc(u_hbm, src_hbm, dst_hbm, zeros_hbm, out_hbm,
                sidx, didx, ring, acc, sem0, sem1, sem2):
    cid = lax.axis_index("c")
    sid = lax.axis_index("s")
    tid = cid * NSUB + sid
    pltpu.sync_copy(zeros_hbm, ring.at[0])
    for w in _WINDOWS:
        pltpu.sync_copy(ring.at[0], acc.at[pl.ds(sid * ROWS_PER_TILE + w, CHUNK)])
    plsc.subcore_barrier()

    sems = (sem0, sem1, sem2)

    def run_segment(h):
        hb = tid * CHUNKS_PER_TILE + h * SEGCH
        pltpu.sync_copy(src_hbm.at[pl.ds(hb, SEGCH)], sidx)
        pltpu.sync_copy(dst_hbm.at[pl.ds(hb, SEGCH)], didx)
        gathers = [
            pltpu.async_copy(u_hbm.at[sidx.at[j]], ring.at[j], sems[j])
            for j in range(NBUF)
        ]
        for i in range(SEGCH):
            p = i % NBUF
            gathers[p].wait()
            pltpu.sync_copy(ring.at[p], acc.at[didx.at[i]], add=True)
            if i + NBUF < SEGCH:
                gathers[p] = pltpu.async_copy(
                    u_hbm.at[sidx.at[i + NBUF]], ring.at[p], sems[p])

    run_segment(0)
    for h in range(1, NSEG):
        @pl.when(cid == FAST_CORE)
        def _(h=h):
            run_segment(h)
    plsc.subcore_barrier()
    for w in _WINDOWS:
        r = sid * ROWS_PER_TILE + w
        pltpu.sync_copy(acc.at[pl.ds(r, CHUNK)], ring.at[0])
        pltpu.sync_copy(ring.at[0], out_hbm.at[pl.ds(cid * ACC_ROWS + r, CHUNK)])


BLK = 1000


def _stage_a_body(x_ref, wfc_ref, bfc_ref, w1_ref, deg_ref, u1_ref, dinv_ref):
    d = deg_ref[...]
    deg = d[0] + d[1] + 1.0

    dinvb = jnp.broadcast_to(lax.rsqrt(deg[:, 0:1]), (BLK, D))
    h0 = jnp.maximum(
        jnp.dot(x_ref[...], wfc_ref[...], preferred_element_type=jnp.float32)
        + bfc_ref[...], 0.0)
    u1_ref[...] = jnp.dot(h0, w1_ref[...],
                          preferred_element_type=jnp.float32) * dinvb
    dinv_ref[...] = dinvb


_stage_a = pl.pallas_call(
    _stage_a_body,
    grid=(N // BLK,),
    in_specs=[
        pl.BlockSpec((BLK, D), lambda i: (i, 0)),
        pl.BlockSpec((D, D), lambda i: (0, 0)),
        pl.BlockSpec((1, D), lambda i: (0, 0)),
        pl.BlockSpec((D, D), lambda i: (0, 0)),
        pl.BlockSpec((NCORES, BLK, DEG_W), lambda i: (0, i, 0)),
    ],
    out_specs=[pl.BlockSpec((BLK, D), lambda i: (i, 0))] * 2,
    out_shape=[jax.ShapeDtypeStruct((N, D), jnp.float32)] * 2,
)


def _stage_b_body(s_ref, u1_ref, dinv_ref, b1_ref, w2_ref, u2_ref):
    s = s_ref[...]
    dinvb = dinv_ref[...]
    h1 = jnp.maximum((s[0] + s[1] + u1_ref[...]) * dinvb + b1_ref[...], 0.0)
    u2_ref[...] = jnp.dot(h1, w2_ref[...],
                          preferred_element_type=jnp.float32) * dinvb


_stage_b = pl.pallas_call(
    _stage_b_body,
    grid=(N // BLK,),
    in_specs=[
        pl.BlockSpec((NCORES, BLK, D), lambda i: (0, i, 0)),
        pl.BlockSpec((BLK, D), lambda i: (i, 0)),
        pl.BlockSpec((BLK, D), lambda i: (i, 0)),
        pl.BlockSpec((1, D), lambda i: (0, 0)),
        pl.BlockSpec((D, D), lambda i: (0, 0)),
    ],
    out_specs=pl.BlockSpec((BLK, D), lambda i: (i, 0)),
    out_shape=jax.ShapeDtypeStruct((N, D), jnp.float32),
)


def _stage_c_body(s_ref, u2_ref, dinv_ref, b2_ref, out_ref):
    s = s_ref[...]
    out_ref[...] = (s[0] + s[1] + u2_ref[...]) * dinv_ref[...] + b2_ref[...]


_stage_c = pl.pallas_call(
    _stage_c_body,
    grid=(N // BLK,),
    in_specs=[
        pl.BlockSpec((NCORES, BLK, D), lambda i: (0, i, 0)),
        pl.BlockSpec((BLK, D), lambda i: (i, 0)),
        pl.BlockSpec((BLK, D), lambda i: (i, 0)),
        pl.BlockSpec((1, D), lambda i: (0, 0)),
    ],
    out_specs=pl.BlockSpec((BLK, D), lambda i: (i, 0)),
    out_shape=jax.ShapeDtypeStruct((N, D), jnp.float32),
)


def kernel(x, edge_index, W_fc, b_fc, W1, b1, W2, b2):
    src = edge_index[0].astype(jnp.int32)
    dst = edge_index[1].astype(jnp.int32)
    pad = SMALL_SLOTS - E_SMALL
    dum = N + (jnp.arange(pad, dtype=jnp.int32) % NSUB)

    def layout(idx, pad_val):
        big = idx[:E_BIG].reshape(NSUB, CHUNKS_PER_TILE, CHUNK)
        small = jnp.concatenate([idx[E_BIG:], pad_val])
        small = small.reshape(NSUB, SEGCH, CHUNK)
        small = jnp.concatenate(
            [small, jnp.zeros((NSUB, CHUNKS_PER_TILE - SEGCH, CHUNK),
                              jnp.int32)], axis=1)
        halves = (big, small) if FAST_CORE == 0 else (small, big)
        return jnp.concatenate(halves, axis=0).reshape(
            NTILES * CHUNKS_PER_TILE, CHUNK)

    src_p = layout(src, jnp.zeros((pad,), jnp.int32))
    dst_p = layout(dst, dum)
    ones128 = jnp.ones((CHUNK, DEG_W), jnp.float32)
    zeros128 = jnp.zeros((CHUNK, D), jnp.float32)

    deg = _degree_sc(dst_p, ones128, zeros128).reshape(NCORES, ACC_ROWS, DEG_W)
    u1, dinvb = _stage_a(x, W_fc, b_fc.reshape(1, D), W1, deg)
    s1 = _scatter_sc(u1, src_p, dst_p, zeros128).reshape(NCORES, ACC_ROWS, D)
    u2 = _stage_b(s1, u1, dinvb, b1.reshape(1, D), W2)
    s2 = _scatter_sc(u2, src_p, dst_p, zeros128).reshape(NCORES, ACC_ROWS, D)
    out = _stage_c(s2, u2, dinvb, b2.reshape(1, D))
    return out


# speedup vs baseline: 15.1174x; 1.0196x over previous
import functools

import jax
import jax.numpy as jnp
from jax import lax
from jax.experimental import pallas as pl
from jax.experimental.pallas import tpu as pltpu
from jax.experimental.pallas import tpu_sc as plsc

N = 10000
D = 128
E = 320000
NCORES = 2
NSUB = 16
NTILES = NCORES * NSUB
CHUNK = 64
CHUNKS_PER_TILE = 256
NSEG = 4
SEGCH = CHUNKS_PER_TILE // NSEG
NBUF = 3
EDGES_PER_TILE = CHUNK * CHUNKS_PER_TILE
E_PAD = EDGES_PER_TILE * NTILES
FAST_CORE = 1
E_BIG = NSUB * EDGES_PER_TILE
E_SMALL = E - E_BIG
SMALL_SLOTS = NSUB * SEGCH * CHUNK
ACC_ROWS = 10112
ROWS_PER_TILE = ACC_ROWS // NSUB
_WINDOWS = tuple(min(k * CHUNK, ROWS_PER_TILE - CHUNK) for k in range(10))
DEG_W = 128

_MESH = plsc.VectorSubcoreMesh(core_axis_name="c", subcore_axis_name="s")


@functools.partial(
    pl.kernel,
    mesh=_MESH,
    out_type=jax.ShapeDtypeStruct((NCORES * ACC_ROWS, DEG_W), jnp.float32),
    scratch_types=[
        pltpu.VMEM((CHUNKS_PER_TILE, CHUNK), jnp.int32),
        pltpu.VMEM((CHUNK, DEG_W), jnp.float32),
        pltpu.VMEM((CHUNK, DEG_W), jnp.float32),
        pltpu.VMEM_SHARED((ACC_ROWS, DEG_W), jnp.float32),
        pltpu.SemaphoreType.DMA,
    ],
)
def _degree_sc(dst_hbm, ones_hbm, zeros_hbm, out_hbm, didx, ones_v, wb_v, acc, sem):
    cid = lax.axis_index("c")
    sid = lax.axis_index("s")
    tid = cid * NSUB + sid
    pltpu.sync_copy(zeros_hbm, wb_v)
    for w in _WINDOWS:
        pltpu.sync_copy(wb_v, acc.at[pl.ds(sid * ROWS_PER_TILE + w, CHUNK)])
    pltpu.sync_copy(ones_hbm, ones_v)
    pltpu.sync_copy(dst_hbm.at[pl.ds(tid * CHUNKS_PER_TILE, CHUNKS_PER_TILE)], didx)
    plsc.subcore_barrier()

    GROUP = 8

    def run_groups(lo, hi):
        for g in range(lo, hi):
            descs = [
                pltpu.async_copy(ones_v, acc.at[didx.at[g * GROUP + j]], sem,
                                 add=True)
                for j in range(GROUP)
            ]
            for desc in descs:
                desc.wait()

    run_groups(0, SEGCH // GROUP)
    @pl.when(cid == FAST_CORE)
    def _():
        run_groups(SEGCH // GROUP, CHUNKS_PER_TILE // GROUP)
    plsc.subcore_barrier()
    for w in _WINDOWS:
        r = sid * ROWS_PER_TILE + w
        pltpu.sync_copy(acc.at[pl.ds(r, CHUNK)], wb_v)
        pltpu.sync_copy(wb_v, out_hbm.at[pl.ds(cid * ACC_ROWS + r, CHUNK)])


@functools.partial(
    pl.kernel,
    mesh=_MESH,
    out_type=jax.ShapeDtypeStruct((NCORES * ACC_ROWS, D), jnp.float32),
    scratch_types=[
        pltpu.VMEM((SEGCH, CHUNK), jnp.int32),
        pltpu.VMEM((SEGCH, CHUNK), jnp.int32),
        pltpu.VMEM((NBUF, CHUNK, D), jnp.float32),
        pltpu.VMEM_SHARED((ACC_ROWS, D), jnp.float32),
        pltpu.SemaphoreType.DMA,
        pltpu.SemaphoreType.DMA,
        pltpu.SemaphoreType.DMA,
    ],
)
def _scatter_sc(u_hbm, src_hbm, dst_hbm, zeros_hbm, out_hbm,
                sidx, didx, ring, acc, sem0, sem1, sem2):
    cid = lax.axis_index("c")
    sid = lax.axis_index("s")
    tid = cid * NSUB + sid
    pltpu.sync_copy(zeros_hbm, ring.at[0])
    for w in _WINDOWS:
        pltpu.sync_copy(ring.at[0], acc.at[pl.ds(sid * ROWS_PER_TILE + w, CHUNK)])
    plsc.subcore_barrier()

    sems = (sem0, sem1, sem2)

    def run_segment(h):
        hb = tid * CHUNKS_PER_TILE + h * SEGCH
        pltpu.sync_copy(src_hbm.at[pl.ds(hb, SEGCH)], sidx)
        pltpu.sync_copy(dst_hbm.at[pl.ds(hb, SEGCH)], didx)
        gathers = [
            pltpu.async_copy(u_hbm.at[sidx.at[j]], ring.at[j], sems[j])
            for j in range(NBUF)
        ]
        for i in range(SEGCH):
            p = i % NBUF
            gathers[p].wait()
            pltpu.sync_copy(ring.at[p], acc.at[didx.at[i]], add=True)
            if i + NBUF < SEGCH:
                gathers[p] = pltpu.async_copy(
                    u_hbm.at[sidx.at[i + NBUF]], ring.at[p], sems[p])

    run_segment(0)
    for h in range(1, NSEG):
        @pl.when(cid == FAST_CORE)
        def _(h=h):
            run_segment(h)
    plsc.subcore_barrier()
    for w in _WINDOWS:
        r = sid * ROWS_PER_TILE + w
        pltpu.sync_copy(acc.at[pl.ds(r, CHUNK)], ring.at[0])
        pltpu.sync_copy(ring.at[0], out_hbm.at[pl.ds(cid * ACC_ROWS + r, CHUNK)])


BLK = 1000


def _stage_a_body(x_ref, wfc_ref, bfc_ref, w1_ref, deg_ref, u1_ref, dinv_ref):
    d = deg_ref[...]
    deg = d[0] + d[1] + 1.0

    dinvb = jnp.broadcast_to(lax.rsqrt(deg[:, 0:1]), (BLK, D))
    h0 = jnp.maximum(
        jnp.dot(x_ref[...], wfc_ref[...], preferred_element_type=jnp.float32)
        + bfc_ref[...], 0.0)
    u1_ref[...] = jnp.dot(h0, w1_ref[...],
                          preferred_element_type=jnp.float32) * dinvb
    dinv_ref[...] = dinvb


_stage_a = pl.pallas_call(
    _stage_a_body,
    grid=(N // BLK,),
    in_specs=[
        pl.BlockSpec((BLK, D), lambda i: (i, 0)),
        pl.BlockSpec((D, D), lambda i: (0, 0)),
        pl.BlockSpec((1, D), lambda i: (0, 0)),
        pl.BlockSpec((D, D), lambda i: (0, 0)),
        pl.BlockSpec((NCORES, BLK, DEG_W), lambda i: (0, i, 0)),
    ],
    out_specs=[pl.BlockSpec((BLK, D), lambda i: (i, 0))] * 2,
    out_shape=[jax.ShapeDtypeStruct((N, D), jnp.float32)] * 2,
)


def _stage_b_body(s_ref, u1_ref, dinv_ref, b1_ref, w2_ref, u2_ref):
    s = s_ref[...]
    dinvb = dinv_ref[...]
    h1 = jnp.maximum((s[0] + s[1] + u1_ref[...]) * dinvb + b1_ref[...], 0.0)
    u2_ref[...] = jnp.dot(h1, w2_ref[...],
                          preferred_element_type=jnp.float32) * dinvb


_stage_b = pl.pallas_call(
    _stage_b_body,
    grid=(N // BLK,),
    in_specs=[
        pl.BlockSpec((NCORES, BLK, D), lambda i: (0, i, 0)),
        pl.BlockSpec((BLK, D), lambda i: (i, 0)),
        pl.BlockSpec((BLK, D), lambda i: (i, 0)),
        pl.BlockSpec((1, D), lambda i: (0, 0)),
        pl.BlockSpec((D, D), lambda i: (0, 0)),
    ],
    out_specs=pl.BlockSpec((BLK, D), lambda i: (i, 0)),
    out_shape=jax.ShapeDtypeStruct((N, D), jnp.float32),
)


def _stage_c_body(s_ref, u2_ref, dinv_ref, b2_ref, out_ref):
    s = s_ref[...]
    out_ref[...] = (s[0] + s[1] + u2_ref[...]) * dinv_ref[...] + b2_ref[...]


_stage_c = pl.pallas_call(
    _stage_c_body,
    grid=(N // BLK,),
    in_specs=[
        pl.BlockSpec((NCORES, BLK, D), lambda i: (0, i, 0)),
        pl.BlockSpec((BLK, D), lambda i: (i, 0)),
        pl.BlockSpec((BLK, D), lambda i: (i, 0)),
        pl.BlockSpec((1, D), lambda i: (0, 0)),
    ],
    out_specs=pl.BlockSpec((BLK, D), lambda i: (i, 0)),
    out_shape=jax.ShapeDtypeStruct((N, D), jnp.float32),
)


def kernel(x, edge_index, W_fc, b_fc, W1, b1, W2, b2):
    src = edge_index[0].astype(jnp.int32)
    dst = edge_index[1].astype(jnp.int32)
    pad = SMALL_SLOTS - E_SMALL
    dum = N + (jnp.arange(pad, dtype=jnp.int32) % NSUB)

    def layout(idx, pad_val):
        big = idx[:E_BIG].reshape(NSUB, CHUNKS_PER_TILE, CHUNK)
        small = jnp.concatenate([idx[E_BIG:], pad_val])
        small = small.reshape(NSUB, SEGCH, CHUNK)
        small = jnp.concatenate(
            [small, jnp.zeros((NSUB, CHUNKS_PER_TILE - SEGCH, CHUNK),
                              jnp.int32)], axis=1)
        halves = (big, small) if FAST_CORE == 0 else (small, big)
        return jnp.concatenate(halves, axis=0).reshape(
            NTILES * CHUNKS_PER_TILE, CHUNK)

    src_p = layout(src, jnp.zeros((pad,), jnp.int32))
    dst_p = layout(dst, dum)
    ones128 = jnp.ones((CHUNK, DEG_W), jnp.float32)
    zeros128 = jnp.zeros((CHUNK, D), jnp.float32)

    deg = _degree_sc(dst_p, ones128, zeros128).reshape(NCORES, ACC_ROWS, DEG_W)
    u1, dinvb = _stage_a(x, W_fc, b_fc.reshape(1, D), W1, deg)
    s1 = _scatter_sc(u1, src_p, dst_p, zeros128).reshape(NCORES, ACC_ROWS, D)
    u2 = _stage_b(s1, u1, dinvb, b1.reshape(1, D), W2)
    s2 = _scatter_sc(u2, src_p, dst_p, zeros128).reshape(NCORES, ACC_ROWS, D)
    out = _stage_c(s2, u2, dinvb, b2.reshape(1, D))
    return out
